# baseline (speedup 1.0000x reference)
import jax
import jax.numpy as jnp
from jax import lax
from jax.experimental import pallas as pl
from jax.experimental.pallas import tpu as pltpu
from jax.experimental.pallas import tpu_sc as plsc

_BATCH = 16384
_SEQ = 200
_D = 32
_NC = 2
_NS = 16
_L = 16
_NW = _NC * _NS
_BW = _BATCH // _NW
_SB = 8
_BB = 128
_C = _SB * _BB
_NSG = _SEQ // _SB
_NBG = _BW // _BB
_G = _NSG * _NBG
_DG = _D // 8


def _embed_body(tags_hbm, table_hbm, out_hbm,
                idx0, idx1, rows0, rows1, tiles_v,
                gsem0, gsem1, osem):
    idx = (idx0, idx1)
    rows = (rows0, rows1)
    gsem = (gsem0, gsem1)

    wid = lax.axis_index("s") * _NC + lax.axis_index("c")
    iota = lax.iota(jnp.int32, _L)

    def block_coords(g):
        sg = g // _NBG
        bg = g % _NBG
        return sg * _SB, wid * _NBG + bg

    def load_idx(b, g):
        s0, Bg = block_coords(g)
        pltpu.sync_copy(tags_hbm.at[pl.ds(s0, _SB), pl.ds(Bg * _BB, _BB)],
                        idx[b])
        for i in range(_SB):
            for j in range(_BB // _L):
                sl = pl.ds(j * _L, _L)
                idx[b][i, sl] = idx[b][i, sl] + 1

    def fire_gathers(b):
        for i in range(_SB):
            pltpu.async_copy(
                table_hbm.at[idx[b].at[i]],
                rows[b].at[pl.ds(i * _BB, _BB), :],
                gsem[b],
            )

    def wait_gathers(b):
        for i in range(_SB):
            pltpu.make_async_copy(
                table_hbm.at[idx[b].at[i]],
                rows[b].at[pl.ds(i * _BB, _BB), :],
                gsem[b],
            ).wait()

    def transpose(b):
        def per_i(i, carry):
            rbases = [iota + (i * _BB + blc * _L) for blc in range(_BB // _L)]
            for d in range(_D):
                col = jnp.full((_L,), d, jnp.int32)
                for blc in range(_BB // _L):
                    v = plsc.load_gather(rows[b], [rbases[blc], col])
                    tiles_v[i, d // 8, d % 8, pl.ds(blc * _L, _L)] = v
            return carry

        lax.fori_loop(0, _SB, per_i, 0)

    def out_slice(g):
        s0, Bg = block_coords(g)
        return out_hbm.at[pl.ds(s0, _SB), :, Bg, :, :]

    def fire_out(g):
        pltpu.async_copy(tiles_v, out_slice(g), osem)

    def wait_out(g):
        pltpu.make_async_copy(tiles_v, out_slice(g), osem).wait()

    def body(g, p):
        wait_gathers(p)
        load_idx(1 - p, g + 1)
        fire_gathers(1 - p)
        wait_out(g - 1)
        transpose(p)
        fire_out(g)

    load_idx(0, 0)
    fire_gathers(0)
    wait_gathers(0)
    load_idx(1, 1)
    fire_gathers(1)
    transpose(0)
    fire_out(0)

    def pair(k, carry):
        g0 = 2 * k + 1
        body(g0, 1)
        body(g0 + 1, 0)
        return carry

    lax.fori_loop(0, (_G - 2) // 2, pair, 0)

    wait_gathers(1)
    wait_out(_G - 2)
    transpose(1)
    fire_out(_G - 1)
    wait_out(_G - 1)


@jax.jit
def kernel(tags, table):
    mesh = plsc.VectorSubcoreMesh(
        core_axis_name="c", subcore_axis_name="s",
        num_cores=_NC, num_subcores=_NS,
    )
    out5 = pl.kernel(
        _embed_body,
        out_type=jax.ShapeDtypeStruct(
            (_SEQ, _DG, _BATCH // _BB, 8, _BB), jnp.float32),
        mesh=mesh,
        scratch_types=[
            pltpu.VMEM((_SB, _BB), jnp.int32),
            pltpu.VMEM((_SB, _BB), jnp.int32),
            pltpu.VMEM((_C, _D), jnp.float32),
            pltpu.VMEM((_C, _D), jnp.float32),
            pltpu.VMEM((_SB, _DG, 8, _BB), jnp.float32),
            pltpu.SemaphoreType.DMA,
            pltpu.SemaphoreType.DMA,
            pltpu.SemaphoreType.DMA,
        ],
        compiler_params=pltpu.CompilerParams(
            use_tc_tiling_on_sc=False, needs_layout_passes=False),
    )(tags.T, table)
    return out5.transpose(2, 4, 0, 1, 3).reshape(_BATCH, _SEQ, _D)


# speedup vs baseline: 1.7312x; 1.7312x over previous
import jax
import jax.numpy as jnp
from jax import lax
from jax.experimental import pallas as pl
from jax.experimental.pallas import tpu as pltpu
from jax.experimental.pallas import tpu_sc as plsc

_BATCH = 16384
_SEQ = 200
_D = 32
_NC = 2
_NS = 16
_L = 16
_NW = _NC * _NS
_BW = _BATCH // _NW
_SB = 8
_BB = 128
_C = _SB * _BB
_NSG = _SEQ // _SB
_NBG = _BW // _BB
_G = _NSG * _NBG
_DG = _D // 8
_LP = _BB + 1


def _embed_body(tags_hbm, table_hbm, out_hbm,
                idx0, idx1, rows0, rows1, tiles_v,
                gsem0, gsem1, osem):
    idx = (idx0, idx1)
    rows = (rows0, rows1)
    gsem = (gsem0, gsem1)

    wid = lax.axis_index("s") * _NC + lax.axis_index("c")
    iota = lax.iota(jnp.int32, _L)

    def block_coords(g):
        sg = g // _NBG
        bg = g % _NBG
        return sg * _SB, wid * _NBG + bg

    def load_idx(b, g):
        s0, Bg = block_coords(g)
        pltpu.sync_copy(tags_hbm.at[pl.ds(s0, _SB), pl.ds(Bg * _BB, _BB)],
                        idx[b])
        for i in range(_SB):
            for j in range(_BB // _L):
                sl = pl.ds(j * _L, _L)
                idx[b][i, sl] = idx[b][i, sl] + 1

    def fire_gathers(b):
        for i in range(_SB):
            pltpu.async_copy(
                table_hbm.at[idx[b].at[i]],
                rows[b].at[pl.ds(i * _BB, _BB), :],
                gsem[b],
            )

    def wait_gathers(b):
        for i in range(_SB):
            pltpu.make_async_copy(
                table_hbm.at[idx[b].at[i]],
                rows[b].at[pl.ds(i * _BB, _BB), :],
                gsem[b],
            ).wait()

    dg_idx = [(lax.iota(jnp.int32, _L) + d0) // 8 for d0 in (0, _L)]
    ds_idx = [(lax.iota(jnp.int32, _L) + d0) % 8 for d0 in (0, _L)]

    def transpose(b):
        def per_i(i, carry):
            i_idx = jnp.full((_L,), 0, jnp.int32) + i
            for bl in range(_BB):
                r = i * _BB + bl
                bl_idx = jnp.full((_L,), bl, jnp.int32)
                for h in range(2):
                    v = rows[b][r, pl.ds(h * _L, _L)]
                    plsc.store_scatter(
                        tiles_v, [i_idx, dg_idx[h], ds_idx[h], bl_idx], v)
            return carry

        lax.fori_loop(0, _SB, per_i, 0)

    def out_slice(g):
        s0, Bg = block_coords(g)
        return out_hbm.at[pl.ds(s0, _SB), :, Bg, :, :]

    def fire_out(g):
        pltpu.async_copy(tiles_v.at[:, :, :, pl.ds(0, _BB)], out_slice(g), osem)

    def wait_out(g):
        pltpu.make_async_copy(tiles_v.at[:, :, :, pl.ds(0, _BB)],
                              out_slice(g), osem).wait()

    def body(g, p):
        wait_gathers(p)
        load_idx(1 - p, g + 1)
        fire_gathers(1 - p)
        wait_out(g - 1)
        transpose(p)
        fire_out(g)

    load_idx(0, 0)
    fire_gathers(0)
    wait_gathers(0)
    load_idx(1, 1)
    fire_gathers(1)
    transpose(0)
    fire_out(0)

    def pair(k, carry):
        g0 = 2 * k + 1
        body(g0, 1)
        body(g0 + 1, 0)
        return carry

    lax.fori_loop(0, (_G - 2) // 2, pair, 0)

    wait_gathers(1)
    wait_out(_G - 2)
    transpose(1)
    fire_out(_G - 1)
    wait_out(_G - 1)


@jax.jit
def kernel(tags, table):
    mesh = plsc.VectorSubcoreMesh(
        core_axis_name="c", subcore_axis_name="s",
        num_cores=_NC, num_subcores=_NS,
    )
    out5 = pl.kernel(
        _embed_body,
        out_type=jax.ShapeDtypeStruct(
            (_SEQ, _DG, _BATCH // _BB, 8, _BB), jnp.float32),
        mesh=mesh,
        scratch_types=[
            pltpu.VMEM((_SB, _BB), jnp.int32),
            pltpu.VMEM((_SB, _BB), jnp.int32),
            pltpu.VMEM((_C, _D), jnp.float32),
            pltpu.VMEM((_C, _D), jnp.float32),
            pltpu.VMEM((_SB, _DG, 8, _LP), jnp.float32),
            pltpu.SemaphoreType.DMA,
            pltpu.SemaphoreType.DMA,
            pltpu.SemaphoreType.DMA,
        ],
        compiler_params=pltpu.CompilerParams(
            use_tc_tiling_on_sc=False, needs_layout_passes=False),
    )(tags.T, table)
    return out5.transpose(2, 4, 0, 1, 3).reshape(_BATCH, _SEQ, _D)
